# baseline (speedup 1.0000x reference)
import functools

import numpy as np
import jax
import jax.numpy as jnp
from jax import lax
from jax.experimental import pallas as pl
from jax.experimental.pallas import tpu as pltpu
from jax.experimental.pallas import tpu_sc as plsc

NB = 18
NBP = 32
FP = 8
FA = 8
T = 512
NW = 32
L = 16
CHUNKS = T // L
NSP = 2
NBS = NBP * NSP
SUMS_W = FP * FA * NBP
ACC_W = FP * FA * NBS
CAC_W = FP * NBS

_C5 = (0.04342837, -0.4048623, 1.5938846, -3.492466, 5.046853, -2.7868056)
_LN2 = 0.6931471805599453
_EYE = np.eye(L, dtype=np.float32)


def _vlog(x):
    bits = plsc.bitcast(x, jnp.int32)
    e = lax.shift_right_arithmetic(bits, 23) - 127
    mbits = lax.bitwise_or(lax.bitwise_and(bits, 0x007FFFFF), 0x3F800000)
    m = plsc.bitcast(mbits, jnp.float32)
    acc = jnp.full((L,), _C5[0], jnp.float32)
    for c in _C5[1:]:
        acc = acc * m + jnp.float32(c)
    return (acc + e.astype(jnp.float32)) * jnp.float32(_LN2)


def _sc_mi_body(pha_hbm, amp_hbm, cut_hbm, out_hbm,
                pha_v, amp_v, cut_v, sums_v, cnts_v, means_v, mi_v, sem):
    core = lax.axis_index("c")
    sub = lax.axis_index("s")
    wid = sub * 2 + core
    bc = wid % 16
    seg = wid // 16

    copies = [pltpu.async_copy(cut_hbm, cut_v, sem)]
    for fp in range(FP):
        copies.append(pltpu.async_copy(
            pha_hbm.at[bc * FP + fp, seg], pha_v.at[pl.ds(fp * T, T)], sem))
    for fa in range(FA):
        copies.append(pltpu.async_copy(
            amp_hbm.at[bc * FA + fa, seg], amp_v.at[pl.ds(fa * T, T)], sem))

    zf = jnp.zeros((L,), jnp.float32)

    def zero_sums(i, c):
        base = i * (8 * L)
        for q in range(8):
            sums_v[pl.ds(base + q * L, L)] = zf
        return c

    lax.fori_loop(0, ACC_W // (8 * L), zero_sums, 0)
    for q in range(CAC_W // L):
        cnts_v[pl.ds(q * L, L)] = zf

    for cp in copies:
        cp.wait()

    ones_f = jnp.ones((L,), jnp.float32)
    ones_i = jnp.ones((L,), jnp.int32)
    zero_i = jnp.zeros((L,), jnp.int32)
    pi = jnp.float32(np.pi)
    inv = jnp.float32(NB / (2.0 * np.pi))
    lane = lax.iota(jnp.int32, L)
    par = lax.bitwise_and(lane, 1)

    def one_chunk(t0):
        avecs = [amp_v[pl.ds(fa * T + t0, L)] for fa in range(FA)]
        for fp in range(FP):
            p = pha_v[pl.ds(fp * T + t0, L)]
            y = (p + pi) * inv + jnp.float32(32.5)
            y = jnp.minimum(jnp.maximum(y, jnp.float32(31.0)), jnp.float32(50.5))
            j = jnp.minimum(jnp.maximum(y.astype(jnp.int32) - 32, 0), NB)
            cj = plsc.load_gather(cut_v, [j])
            u = j + jnp.where(cj < p, ones_i, zero_i)
            b = jnp.minimum(jnp.maximum(u - 1, 0), NB - 1)
            b2 = b + b + par
            plsc.addupdate_scatter(cnts_v, [fp * NBS + b2], ones_f)
            fbase = fp * (FA * NBS) + b2
            for fa in range(FA):
                plsc.addupdate_scatter(sums_v, [fbase + fa * NBS], avecs[fa])

    @plsc.parallel_loop(0, CHUNKS, 1, unroll=2)
    def _hist(ci):
        one_chunk(ci * L)

    eps = jnp.float32(1e-9)
    i2 = lane + lane
    for fp in range(FP):
        cb = fp * NBS
        ce0 = (plsc.load_gather(cnts_v, [i2 + cb])
               + plsc.load_gather(cnts_v, [i2 + (cb + 1)]) + eps)
        ce1 = (plsc.load_gather(cnts_v, [i2 + (cb + 2 * L)])
               + plsc.load_gather(cnts_v, [i2 + (cb + 2 * L + 1)]) + eps)
        for fa in range(FA):
            sb = (fp * FA + fa) * NBS
            s0 = (plsc.load_gather(sums_v, [i2 + sb])
                  + plsc.load_gather(sums_v, [i2 + (sb + 1)]))
            s1 = (plsc.load_gather(sums_v, [i2 + (sb + 2 * L)])
                  + plsc.load_gather(sums_v, [i2 + (sb + 2 * L + 1)]))
            base = (fp * FA + fa) * NBP
            means_v[pl.ds(base, L)] = s0 / ce0
            means_v[pl.ds(base + L, L)] = s1 / ce1
    lane32 = lax.iota(jnp.int32, L) * NBP
    for g in range(FP * FA // L):
        gb = g * (L * NBP)
        rs = zf
        for bn in range(NB):
            rs = rs + plsc.load_gather(means_v, [lane32 + (gb + bn)])
        rse = rs + eps
        acc = zf
        for bn in range(NB):
            m = plsc.load_gather(means_v, [lane32 + (gb + bn)])
            pr = m / rse
            acc = acc + pr * _vlog(pr + eps)
        mi_v[pl.ds(g * L, L)] = acc

    pltpu.sync_copy(mi_v, out_hbm.at[wid])


@functools.cache
def _sc_mi():
    mesh = plsc.VectorSubcoreMesh(core_axis_name="c", subcore_axis_name="s")
    return pl.kernel(
        _sc_mi_body,
        out_type=jax.ShapeDtypeStruct((NW, FP * FA), jnp.float32),
        mesh=mesh,
        compiler_params=pltpu.CompilerParams(needs_layout_passes=False),
        scratch_types=(pltpu.VMEM((FP * T,), jnp.float32),
                       pltpu.VMEM((FA * T,), jnp.float32),
                       pltpu.VMEM((32,), jnp.float32),
                       pltpu.VMEM((ACC_W,), jnp.float32),
                       pltpu.VMEM((CAC_W,), jnp.float32),
                       pltpu.VMEM((SUMS_W,), jnp.float32),
                       pltpu.VMEM((FP * FA,), jnp.float32),
                       pltpu.SemaphoreType.DMA),
    )


_LOG_NUM = float(np.log(np.float32(NB) + np.float32(1e-9)))
_LOG_DEN = float(np.log(np.float32(NB)))


def _tc_body(e_ref, out_ref):
    e = (e_ref[0:16] + e_ref[16:NW]) * jnp.float32(0.5)
    out_ref[...] = (jnp.float32(_LOG_NUM) + e) / jnp.float32(_LOG_DEN)


def kernel(pha, amp):
    pha = pha.astype(jnp.float32)
    amp = amp.astype(jnp.float32)
    phat = pha.reshape(2 * 8 * FP, 2, T)
    ampt = amp.reshape(2 * 8 * FA, 2, T)
    cut = jnp.linspace(-np.pi, np.pi, NB + 1).astype(jnp.float32)
    cutp = jnp.concatenate([cut, jnp.full((32 - (NB + 1),), 1e30, jnp.float32)])

    ent = _sc_mi()(phat, ampt, cutp)

    mi = pl.pallas_call(
        _tc_body,
        out_shape=jax.ShapeDtypeStruct((16, FP * FA), jnp.float32),
    )(ent)

    return mi.reshape(2, 8, FP, FA)


# speedup vs baseline: 1.0640x; 1.0640x over previous
import functools

import numpy as np
import jax
import jax.numpy as jnp
from jax import lax
from jax.experimental import pallas as pl
from jax.experimental.pallas import tpu as pltpu
from jax.experimental.pallas import tpu_sc as plsc

NB = 18
NBP = 32
FP = 8
FA = 8
T = 512
NW = 32
L = 16
CHUNKS = T // L
NSP = 2
NBS = NBP * NSP
SUMS_W = FP * FA * NBP
ACC_W = FP * FA * NBS
CAC_W = FP * NBS

_C5 = (0.04342837, -0.4048623, 1.5938846, -3.492466, 5.046853, -2.7868056)
_LN2 = 0.6931471805599453
_EYE = np.eye(L, dtype=np.float32)


def _vlog(x):
    bits = plsc.bitcast(x, jnp.int32)
    e = lax.shift_right_arithmetic(bits, 23) - 127
    mbits = lax.bitwise_or(lax.bitwise_and(bits, 0x007FFFFF), 0x3F800000)
    m = plsc.bitcast(mbits, jnp.float32)
    acc = jnp.full((L,), _C5[0], jnp.float32)
    for c in _C5[1:]:
        acc = acc * m + jnp.float32(c)
    return (acc + e.astype(jnp.float32)) * jnp.float32(_LN2)


def _sc_mi_body(pha_hbm, amp_hbm, cut_hbm, out_hbm,
                pha_v, amp_v, cut_v, sums_v, cnts_v, means_v, mi_v, sem):
    core = lax.axis_index("c")
    sub = lax.axis_index("s")
    wid = sub * 2 + core
    bc = wid % 16
    seg = wid // 16

    copies = [
        pltpu.async_copy(cut_hbm, cut_v, sem),
        pltpu.async_copy(pha_hbm.at[pl.ds(bc * FP, FP), seg], pha_v, sem),
        pltpu.async_copy(amp_hbm.at[pl.ds(bc * FA, FA), seg], amp_v, sem),
    ]

    zf = jnp.zeros((L,), jnp.float32)

    def zero_sums(i, c):
        base = i * (8 * L)
        for q in range(8):
            sums_v[pl.ds(base + q * L, L)] = zf
        return c

    lax.fori_loop(0, ACC_W // (8 * L), zero_sums, 0)
    for q in range(CAC_W // L):
        cnts_v[pl.ds(q * L, L)] = zf

    for cp in copies:
        cp.wait()

    ones_f = jnp.ones((L,), jnp.float32)
    ones_i = jnp.ones((L,), jnp.int32)
    zero_i = jnp.zeros((L,), jnp.int32)
    pi = jnp.float32(np.pi)
    inv = jnp.float32(NB / (2.0 * np.pi))
    lane = lax.iota(jnp.int32, L)
    par = lax.bitwise_and(lane, 1)

    def one_chunk(t0):
        avecs = [amp_v[fa, pl.ds(t0, L)] for fa in range(FA)]
        for fp in range(FP):
            p = pha_v[fp, pl.ds(t0, L)]
            y = (p + pi) * inv + jnp.float32(32.5)
            y = jnp.minimum(jnp.maximum(y, jnp.float32(31.0)), jnp.float32(50.5))
            j = jnp.minimum(jnp.maximum(y.astype(jnp.int32) - 32, 0), NB)
            cj = plsc.load_gather(cut_v, [j])
            u = j + jnp.where(cj < p, ones_i, zero_i)
            b = jnp.minimum(jnp.maximum(u - 1, 0), NB - 1)
            b2 = b + b + par
            plsc.addupdate_scatter(cnts_v, [fp * NBS + b2], ones_f)
            fbase = fp * (FA * NBS) + b2
            for fa in range(FA):
                plsc.addupdate_scatter(sums_v, [fbase + fa * NBS], avecs[fa])

    def chunk(ci, c):
        t0 = ci * (2 * L)
        one_chunk(t0)
        one_chunk(t0 + L)
        return c

    lax.fori_loop(0, CHUNKS // 2, chunk, 0)

    eps = jnp.float32(1e-9)
    i2 = lane + lane
    for fp in range(FP):
        cb = fp * NBS
        ce0 = (plsc.load_gather(cnts_v, [i2 + cb])
               + plsc.load_gather(cnts_v, [i2 + (cb + 1)]) + eps)
        ce1 = (plsc.load_gather(cnts_v, [i2 + (cb + 2 * L)])
               + plsc.load_gather(cnts_v, [i2 + (cb + 2 * L + 1)]) + eps)
        for fa in range(FA):
            sb = (fp * FA + fa) * NBS
            s0 = (plsc.load_gather(sums_v, [i2 + sb])
                  + plsc.load_gather(sums_v, [i2 + (sb + 1)]))
            s1 = (plsc.load_gather(sums_v, [i2 + (sb + 2 * L)])
                  + plsc.load_gather(sums_v, [i2 + (sb + 2 * L + 1)]))
            base = (fp * FA + fa) * NBP
            means_v[pl.ds(base, L)] = s0 / ce0
            means_v[pl.ds(base + L, L)] = s1 / ce1
    lane32 = lax.iota(jnp.int32, L) * NBP
    for g in range(FP * FA // L):
        gb = g * (L * NBP)
        rs = zf
        for bn in range(NB):
            rs = rs + plsc.load_gather(means_v, [lane32 + (gb + bn)])
        rse = rs + eps
        acc = zf
        for bn in range(NB):
            m = plsc.load_gather(means_v, [lane32 + (gb + bn)])
            pr = m / rse
            acc = acc + pr * _vlog(pr + eps)
        mi_v[pl.ds(g * L, L)] = acc

    pltpu.sync_copy(mi_v, out_hbm.at[wid])


@functools.cache
def _sc_mi():
    mesh = plsc.VectorSubcoreMesh(core_axis_name="c", subcore_axis_name="s")
    return pl.kernel(
        _sc_mi_body,
        out_type=jax.ShapeDtypeStruct((NW, FP * FA), jnp.float32),
        mesh=mesh,
        compiler_params=pltpu.CompilerParams(needs_layout_passes=False),
        scratch_types=(pltpu.VMEM((FP, T), jnp.float32),
                       pltpu.VMEM((FA, T), jnp.float32),
                       pltpu.VMEM((32,), jnp.float32),
                       pltpu.VMEM((ACC_W,), jnp.float32),
                       pltpu.VMEM((CAC_W,), jnp.float32),
                       pltpu.VMEM((SUMS_W,), jnp.float32),
                       pltpu.VMEM((FP * FA,), jnp.float32),
                       pltpu.SemaphoreType.DMA),
    )


_LOG_NUM = float(np.log(np.float32(NB) + np.float32(1e-9)))
_LOG_DEN = float(np.log(np.float32(NB)))


def _tc_body(e_ref, out_ref):
    e = (e_ref[0:16] + e_ref[16:NW]) * jnp.float32(0.5)
    out_ref[...] = (jnp.float32(_LOG_NUM) + e) / jnp.float32(_LOG_DEN)


def kernel(pha, amp):
    pha = pha.astype(jnp.float32)
    amp = amp.astype(jnp.float32)
    phat = pha.reshape(2 * 8 * FP, 2, T)
    ampt = amp.reshape(2 * 8 * FA, 2, T)
    cut = jnp.linspace(-np.pi, np.pi, NB + 1).astype(jnp.float32)
    cutp = jnp.concatenate([cut, jnp.full((32 - (NB + 1),), 1e30, jnp.float32)])

    ent = _sc_mi()(phat, ampt, cutp)

    mi = pl.pallas_call(
        _tc_body,
        out_shape=jax.ShapeDtypeStruct((16, FP * FA), jnp.float32),
    )(ent)

    return mi.reshape(2, 8, FP, FA)
